# speedup vs baseline: 6.6263x; 6.6263x over previous
import functools

import jax
import jax.numpy as jnp
from jax import lax
from jax.experimental import pallas as pl
from jax.experimental.pallas import tpu as pltpu
from jax.experimental.pallas import tpu_sc as plsc

_HID = 128


@functools.lru_cache(maxsize=None)
def _sc_gather(n_rows: int, vocab: int, chunk: int):
    info = plsc.get_sparse_core_info()
    nc, ns = info.num_cores, info.num_subcores
    nw = nc * ns
    rows_per_w = n_rows // nw
    n_chunks = rows_per_w // chunk

    mesh = plsc.VectorSubcoreMesh(core_axis_name="c", subcore_axis_name="s")

    @functools.partial(
        pl.kernel,
        mesh=mesh,
        out_type=jax.ShapeDtypeStruct((n_rows, _HID), jnp.float32),
        scratch_types=[
            pltpu.VMEM((chunk,), jnp.int32),
            pltpu.VMEM((chunk, _HID), jnp.float32),
            pltpu.SemaphoreType.DMA,
        ],
    )
    def k(table_hbm, ids_hbm, out_hbm, idx_v, rows_v, sem):
        wid = lax.axis_index("s") * nc + lax.axis_index("c")
        base = wid * rows_per_w

        def body(i, carry):
            off = base + i * chunk
            pltpu.sync_copy(ids_hbm.at[pl.ds(off, chunk)], idx_v)
            pltpu.async_copy(table_hbm.at[idx_v], rows_v, sem).wait()
            pltpu.sync_copy(rows_v, out_hbm.at[pl.ds(off, chunk)])
            return carry

        lax.fori_loop(0, n_chunks, body, 0)

    return k


def _ln_body(x_ref, pos_ref, tt_ref, t0_ref, dlt_ref, g_ref, b_ref, o_ref):
    x = x_ref[...]
    x = (
        x
        + pos_ref[...][:, None, :]
        + t0_ref[...][None, None, :]
        + tt_ref[...][..., None] * dlt_ref[...][None, None, :]
    )
    mean = jnp.mean(x, axis=-1, keepdims=True)
    xc = x - mean
    var = jnp.mean(xc * xc, axis=-1, keepdims=True)
    o_ref[...] = xc * lax.rsqrt(var + 1e-5) * g_ref[...] + b_ref[...]


def kernel(input_ids, position_ids, token_type_ids, word_emb, pos_emb,
           type_emb, ln_gamma, ln_beta):
    s, b = input_ids.shape
    h = word_emb.shape[1]
    n_rows = s * b

    ids_flat = input_ids.reshape(-1).astype(jnp.int32)
    gathered = _sc_gather(n_rows, word_emb.shape[0], 128)(word_emb, ids_flat)

    pos_table = jnp.take(pos_emb, position_ids[0].astype(jnp.int32), axis=0)
    tt = token_type_ids.astype(jnp.float32)
    t0 = type_emb[0]
    dlt = type_emb[1] - type_emb[0]

    sblk = 8
    grid = (s // sblk,)
    out = pl.pallas_call(
        _ln_body,
        grid=grid,
        in_specs=[
            pl.BlockSpec((sblk, b, h), lambda i: (i, 0, 0)),
            pl.BlockSpec((sblk, h), lambda i: (i, 0)),
            pl.BlockSpec((sblk, b), lambda i: (i, 0)),
            pl.BlockSpec((h,), lambda i: (0,)),
            pl.BlockSpec((h,), lambda i: (0,)),
            pl.BlockSpec((h,), lambda i: (0,)),
            pl.BlockSpec((h,), lambda i: (0,)),
        ],
        out_specs=pl.BlockSpec((sblk, b, h), lambda i: (i, 0, 0)),
        out_shape=jax.ShapeDtypeStruct((s, b, h), jnp.float32),
    )(gathered.reshape(s, b, h), pos_table, tt, t0, dlt, ln_gamma, ln_beta)
    return out


# speedup vs baseline: 9.0641x; 1.3679x over previous
import functools

import jax
import jax.numpy as jnp
from jax import lax
from jax.experimental import pallas as pl
from jax.experimental.pallas import tpu as pltpu
from jax.experimental.pallas import tpu_sc as plsc

_HID = 128


@functools.lru_cache(maxsize=None)
def _sc_gather(n_rows: int, vocab: int, chunk: int):
    info = plsc.get_sparse_core_info()
    nc, ns = info.num_cores, info.num_subcores
    nw = nc * ns
    rows_per_w = n_rows // nw
    n_chunks = rows_per_w // chunk

    mesh = plsc.VectorSubcoreMesh(core_axis_name="c", subcore_axis_name="s")

    @functools.partial(
        pl.kernel,
        mesh=mesh,
        out_type=jax.ShapeDtypeStruct((n_rows, _HID), jnp.float32),
        scratch_types=[
            pltpu.VMEM((n_chunks, chunk), jnp.int32),
            pltpu.VMEM((chunk, _HID), jnp.float32),
            pltpu.VMEM((chunk, _HID), jnp.float32),
            pltpu.SemaphoreType.DMA,
            pltpu.SemaphoreType.DMA,
            pltpu.SemaphoreType.DMA,
            pltpu.SemaphoreType.DMA,
        ],
    )
    def k(table_hbm, ids_hbm, out_hbm, idx_all, buf0, buf1, g0, g1, s0, s1):
        wid = lax.axis_index("s") * nc + lax.axis_index("c")
        base = wid * rows_per_w
        buf = (buf0, buf1)
        gs = (g0, g1)
        ss = (s0, s1)

        pltpu.sync_copy(ids_hbm.at[wid], idx_all)
        pltpu.async_copy(table_hbm.at[idx_all.at[0]], buf0, g0)

        def pair(p, carry):
            for q in range(2):
                i = 2 * p + q
                cur, nxt = q, 1 - q

                @pl.when(i + 1 < n_chunks)
                def _fire():
                    @pl.when(i >= 1)
                    def _drain():
                        pltpu.make_async_copy(
                            buf[nxt], out_hbm.at[pl.ds(base, chunk)], ss[nxt]
                        ).wait()

                    pltpu.async_copy(
                        table_hbm.at[idx_all.at[i + 1]], buf[nxt], gs[nxt]
                    )

                pltpu.make_async_copy(
                    table_hbm.at[idx_all.at[i]], buf[cur], gs[cur]
                ).wait()
                pltpu.async_copy(
                    buf[cur], out_hbm.at[pl.ds(base + i * chunk, chunk)], ss[cur]
                )
            return carry

        lax.fori_loop(0, n_chunks // 2, pair, 0)
        for b in range(2):
            pltpu.make_async_copy(
                buf[b], out_hbm.at[pl.ds(base, chunk)], ss[b]
            ).wait()

    return k


def _ln_body(x_ref, pos_ref, tt_ref, t0_ref, dlt_ref, g_ref, b_ref, o_ref):
    x = x_ref[...]
    x = (
        x
        + pos_ref[...][:, None, :]
        + t0_ref[...][None, None, :]
        + tt_ref[...][..., None] * dlt_ref[...][None, None, :]
    )
    mean = jnp.mean(x, axis=-1, keepdims=True)
    xc = x - mean
    var = jnp.mean(xc * xc, axis=-1, keepdims=True)
    o_ref[...] = xc * lax.rsqrt(var + 1e-5) * g_ref[...] + b_ref[...]


def kernel(input_ids, position_ids, token_type_ids, word_emb, pos_emb,
           type_emb, ln_gamma, ln_beta):
    s, b = input_ids.shape
    h = word_emb.shape[1]
    n_rows = s * b

    chunk = 128
    info = plsc.get_sparse_core_info()
    nw = info.num_cores * info.num_subcores
    ids_t = input_ids.reshape(nw, (n_rows // nw) // chunk, chunk).astype(jnp.int32)
    gathered = _sc_gather(n_rows, word_emb.shape[0], chunk)(word_emb, ids_t)

    pos_table = jnp.take(pos_emb, position_ids[0].astype(jnp.int32), axis=0)
    tt = token_type_ids.astype(jnp.float32)
    t0 = type_emb[0]
    dlt = type_emb[1] - type_emb[0]

    sblk = 8
    grid = (s // sblk,)
    out = pl.pallas_call(
        _ln_body,
        grid=grid,
        in_specs=[
            pl.BlockSpec((sblk, b, h), lambda i: (i, 0, 0)),
            pl.BlockSpec((sblk, h), lambda i: (i, 0)),
            pl.BlockSpec((sblk, b), lambda i: (i, 0)),
            pl.BlockSpec((h,), lambda i: (0,)),
            pl.BlockSpec((h,), lambda i: (0,)),
            pl.BlockSpec((h,), lambda i: (0,)),
            pl.BlockSpec((h,), lambda i: (0,)),
        ],
        out_specs=pl.BlockSpec((sblk, b, h), lambda i: (i, 0, 0)),
        out_shape=jax.ShapeDtypeStruct((s, b, h), jnp.float32),
    )(gathered.reshape(s, b, h), pos_table, tt, t0, dlt, ln_gamma, ln_beta)
    return out


# speedup vs baseline: 9.8320x; 1.0847x over previous
import functools

import jax
import jax.numpy as jnp
from jax import lax
from jax.experimental import pallas as pl
from jax.experimental.pallas import tpu as pltpu
from jax.experimental.pallas import tpu_sc as plsc

_HID = 128


@functools.lru_cache(maxsize=None)
def _sc_gather(n_rows: int, chunk: int):
    info = plsc.get_sparse_core_info()
    nc, ns = info.num_cores, info.num_subcores
    nw = nc * ns
    rows_per_w = n_rows // nw
    n_chunks = rows_per_w // chunk

    mesh = plsc.VectorSubcoreMesh(core_axis_name="c", subcore_axis_name="s")

    @functools.partial(
        pl.kernel,
        mesh=mesh,
        out_type=jax.ShapeDtypeStruct((n_rows, _HID), jnp.float32),
        scratch_types=[
            pltpu.VMEM((n_chunks, chunk), jnp.int32),
            pltpu.VMEM((chunk, _HID), jnp.float32),
            pltpu.VMEM((chunk, _HID), jnp.float32),
            pltpu.SemaphoreType.DMA,
            pltpu.SemaphoreType.DMA,
            pltpu.SemaphoreType.DMA,
            pltpu.SemaphoreType.DMA,
        ],
    )
    def k(table_hbm, ids_hbm, out_hbm, idx_all, buf0, buf1, g0, g1, s0, s1):
        wid = lax.axis_index("s") * nc + lax.axis_index("c")
        base = wid * rows_per_w
        buf = (buf0, buf1)
        gs = (g0, g1)
        ss = (s0, s1)

        pltpu.sync_copy(ids_hbm.at[wid], idx_all)
        pltpu.async_copy(table_hbm.at[idx_all.at[0]], buf0, g0)

        def pair(p, carry):
            for q in range(2):
                i = 2 * p + q
                cur, nxt = q, 1 - q

                @pl.when(i + 1 < n_chunks)
                def _fire():
                    @pl.when(i >= 1)
                    def _drain():
                        pltpu.make_async_copy(
                            buf[nxt], out_hbm.at[pl.ds(base, chunk)], ss[nxt]
                        ).wait()

                    pltpu.async_copy(
                        table_hbm.at[idx_all.at[i + 1]], buf[nxt], gs[nxt]
                    )

                pltpu.make_async_copy(
                    table_hbm.at[idx_all.at[i]], buf[cur], gs[cur]
                ).wait()
                pltpu.async_copy(
                    buf[cur], out_hbm.at[pl.ds(base + i * chunk, chunk)], ss[cur]
                )
            return carry

        lax.fori_loop(0, n_chunks // 2, pair, 0)
        for b in range(2):
            pltpu.make_async_copy(
                buf[b], out_hbm.at[pl.ds(base, chunk)], ss[b]
            ).wait()

    return k


def _ln_first(x_ref, c1_ref, tt_ref, dlt_ref, g_ref, b_ref, o_ref):
    _ln_impl(x_ref, c1_ref, tt_ref, dlt_ref, g_ref, b_ref, o_ref)


def _ln_next(x_ref, c1_ref, tt_ref, dlt_ref, g_ref, b_ref, prev_ref, o_ref):
    del prev_ref
    _ln_impl(x_ref, c1_ref, tt_ref, dlt_ref, g_ref, b_ref, o_ref)


def _ln_impl(x_ref, c1_ref, tt_ref, dlt_ref, g_ref, b_ref, o_ref):
    x = (
        x_ref[...]
        + c1_ref[...][:, None, :]
        + tt_ref[...][..., None] * dlt_ref[...][None, None, :]
    )
    mean = jnp.mean(x, axis=-1, keepdims=True)
    xc = x - mean
    var = jnp.mean(xc * xc, axis=-1, keepdims=True)
    o_ref[...] = xc * lax.rsqrt(var + 1e-5) * g_ref[...] + b_ref[...]


def kernel(input_ids, position_ids, token_type_ids, word_emb, pos_emb,
           type_emb, ln_gamma, ln_beta):
    s, b = input_ids.shape
    h = word_emb.shape[1]
    chunk = 128
    sblk = 8
    nsl = 4
    s_sl = s // nsl
    rows_sl = s_sl * b
    info = plsc.get_sparse_core_info()
    nw = info.num_cores * info.num_subcores
    n_chunks = (rows_sl // nw) // chunk

    pos_table = jnp.take(pos_emb, position_ids[0].astype(jnp.int32), axis=0)
    c1 = pos_table + type_emb[0]
    dlt = type_emb[1] - type_emb[0]
    tt = token_type_ids.astype(jnp.float32)

    sc = _sc_gather(rows_sl, chunk)
    out = None
    for k in range(nsl):
        ids_k = (
            input_ids[k * s_sl:(k + 1) * s_sl]
            .reshape(nw, n_chunks, chunk)
            .astype(jnp.int32)
        )
        gath = sc(word_emb, ids_k).reshape(s_sl, b, h)

        common_specs = [
            pl.BlockSpec((sblk, b, h), lambda i: (i, 0, 0)),
            pl.BlockSpec((sblk, h), lambda i: (i, 0)),
            pl.BlockSpec((sblk, b), lambda i: (i, 0)),
            pl.BlockSpec((h,), lambda i: (0,)),
            pl.BlockSpec((h,), lambda i: (0,)),
            pl.BlockSpec((h,), lambda i: (0,)),
        ]
        common_args = (
            gath,
            lax.dynamic_slice_in_dim(c1, k * s_sl, s_sl),
            lax.dynamic_slice_in_dim(tt, k * s_sl, s_sl),
            dlt,
            ln_gamma,
            ln_beta,
        )
        out_spec = pl.BlockSpec(
            (sblk, b, h), lambda i, k=k: (i + k * (s_sl // sblk), 0, 0)
        )
        out_shape = jax.ShapeDtypeStruct((s, b, h), jnp.float32)
        if k == 0:
            out = pl.pallas_call(
                _ln_first,
                grid=(s_sl // sblk,),
                in_specs=common_specs,
                out_specs=out_spec,
                out_shape=out_shape,
            )(*common_args)
        else:
            out = pl.pallas_call(
                _ln_next,
                grid=(s_sl // sblk,),
                in_specs=common_specs + [pl.BlockSpec(memory_space=pl.ANY)],
                out_specs=out_spec,
                out_shape=out_shape,
                input_output_aliases={6: 0},
            )(*common_args, out)
    return out
